# baseline (speedup 1.0000x reference)
import functools

import jax
import jax.numpy as jnp
from jax import lax
from jax.experimental import pallas as pl
from jax.experimental.pallas import tpu as pltpu
from jax.experimental.pallas import tpu_sc as plsc

N = 10000
D = 256
DH = 128
NPAD = 10240
ACC = NPAD + 128
E = 160000
EPT = E // 16
TIR = NPAD // 16
BLK = 128
DBLK = 82
BLKA = 88
NP = 122
DBLKA = NP + 4
RFIRST = 2 * BLKA
DEGW = 16
EPS = 1e-5
BR = 2048
BRS = 2000


def _sc_mesh():
    return plsc.VectorSubcoreMesh(core_axis_name="c", subcore_axis_name="s")


def _deg(dst3, ones):
    @functools.partial(
        pl.kernel,
        out_type=jax.ShapeDtypeStruct((NPAD, DEGW), jnp.float32),
        mesh=_sc_mesh(),
        scratch_types=[
            pltpu.VMEM((DBLK, BLK), jnp.int32),
            pltpu.VMEM((BLK, DEGW), jnp.float32),
            pltpu.VMEM_SHARED((ACC, DEGW), jnp.float32),
            pltpu.SemaphoreType.DMA,
        ],
    )
    def k(dst_hbm, ones_hbm, out_hbm, dsts, ones_v, acc, sem):
        c = lax.axis_index("c")
        s = lax.axis_index("s")
        pltpu.sync_copy(dst_hbm.at[s], dsts)
        pltpu.sync_copy(ones_hbm.at[pl.ds(0, BLK)], ones_v)
        pltpu.sync_copy(ones_hbm.at[pl.ds(s * TIR, TIR)],
                        acc.at[pl.ds(s * TIR, TIR)])
        plsc.subcore_barrier()

        def body(j, carry):
            pltpu.sync_copy(ones_v, acc.at[dsts.at[j]], add=True)
            return carry

        lax.fori_loop(0, DBLK, body, 0)
        plsc.subcore_barrier()
        hh = NPAD // 2
        pltpu.sync_copy(acc.at[pl.ds(c * hh + s * (hh // 16), hh // 16)],
                        out_hbm.at[pl.ds(c * hh + s * (hh // 16), hh // 16)])

    return k(dst3, ones)


def _agg(p2, src3, dst3):
    @functools.partial(
        pl.kernel,
        out_type=jax.ShapeDtypeStruct((2 * NPAD, DH), jnp.float32),
        mesh=_sc_mesh(),
        scratch_types=[
            pltpu.VMEM((8, BLKA), jnp.int32),
            pltpu.VMEM((8, BLKA), jnp.int32),
            pltpu.VMEM((4, BLKA, DH), jnp.float32),
            pltpu.VMEM_SHARED((NPAD, DH), jnp.float32),
            pltpu.SemaphoreType.DMA,
            pltpu.SemaphoreType.DMA,
            pltpu.SemaphoreType.DMA,
            pltpu.SemaphoreType.DMA,
        ],
    )
    def k(p_hbm, src_hbm, dst_hbm, out_hbm, sidx, didx, rows, acc,
          sem_g, sem_s, sem_d, sem_c):
        c = lax.axis_index("c")
        s = lax.axis_index("s")
        w = c * 16 + s
        pltpu.sync_copy(p_hbm.at[pl.ds(c * NPAD + s * TIR, TIR)],
                        acc.at[pl.ds(s * TIR, TIR)])
        plsc.subcore_barrier()

        def load_idx(j, sl):
            pltpu.async_copy(src_hbm.at[w, j], sidx.at[sl], sem_s)
            pltpu.async_copy(dst_hbm.at[s, j], didx.at[sl], sem_d)

        def gather(sl, rb):
            pltpu.async_copy(p_hbm.at[sidx.at[sl]], rows.at[rb], sem_g)

        def scatter(rb, sl):
            pltpu.async_copy(rows.at[rb], acc.at[didx.at[sl]], sem_c,
                             add=True)

        def wait_idx():
            pltpu.make_async_copy(src_hbm.at[0, 0], sidx.at[0], sem_s).wait()
            pltpu.make_async_copy(dst_hbm.at[0, 0], didx.at[0], sem_d).wait()

        def wait_gather():
            pltpu.make_async_copy(p_hbm.at[pl.ds(0, BLKA)], rows.at[0],
                                  sem_g).wait()

        def wait_scatter():
            pltpu.make_async_copy(p_hbm.at[pl.ds(0, BLKA)], rows.at[0],
                                  sem_c).wait()

        for j in range(4):
            pltpu.sync_copy(src_hbm.at[w, j], sidx.at[j])
            pltpu.sync_copy(dst_hbm.at[s, j], didx.at[j])
        gather(0, 0)
        gather(1, 1)
        load_idx(4, 4)
        load_idx(5, 5)
        wait_gather()
        gather(2, 2)
        scatter(0, 0)
        load_idx(6, 6)
        wait_gather()
        gather(3, 3)
        scatter(1, 1)
        load_idx(7, 7)

        def body(kk, carry):
            u0 = 2 + kk * 8
            for t in range(8):
                u = u0 + t
                wait_gather()
                wait_idx()
                wait_scatter()
                gather((u + 2) % 8, (u + 2) % 4)
                scatter(u % 4, u % 8)
                load_idx(u + 4, (u + 4) % 8)
            return carry

        lax.fori_loop(0, (NP - 2) // 8, body, 0)
        for _ in range(2):
            wait_gather()
            wait_idx()
            wait_scatter()
        plsc.subcore_barrier()
        pltpu.sync_copy(acc.at[pl.ds(s * TIR, TIR)],
                        out_hbm.at[pl.ds(c * NPAD + s * TIR, TIR)])

    return k(p2, src3, dst3)


def _pre(x_p, w, degw):
    def body(x_ref, w_ref, dg_ref, o_ref):
        dinv = lax.rsqrt(dg_ref[:, 0:1])
        o_ref[0] = jnp.dot(x_ref[...], w_ref[...],
                           preferred_element_type=jnp.float32) * dinv

    return pl.pallas_call(
        body,
        grid=(2, NPAD // BR),
        in_specs=[
            pl.BlockSpec((BR, D), lambda c, i: (i, 0)),
            pl.BlockSpec((D, DH), lambda c, i: (0, c)),
            pl.BlockSpec((BR, DEGW), lambda c, i: (i, 0)),
        ],
        out_specs=pl.BlockSpec((1, BR, DH), lambda c, i: (c, i, 0)),
        out_shape=jax.ShapeDtypeStruct((2, NPAD, DH), jnp.float32),
    )(x_p, w, degw)


def _bn_apply(a3, degw, b, g, be, w):
    def body(a_ref, dg_ref, b_ref, g_ref, be_ref, w_ref, o_ref, st_ref):
        p = pl.program_id(0)
        i = pl.program_id(1)
        dinv = lax.rsqrt(dg_ref[:, 0:1])
        a = jnp.concatenate([a_ref[0], a_ref[1]], axis=1)
        conv = a * dinv + b_ref[...]
        rid = i * BR + lax.broadcasted_iota(jnp.int32, (BR, 1), 0)
        valid = rid < N

        @pl.when(p == 0)
        def _():
            cm = jnp.where(valid, conv, 0.0)
            s1 = jnp.sum(cm, axis=0, keepdims=True)
            s2 = jnp.sum(cm * cm, axis=0, keepdims=True)
            blk = jnp.concatenate(
                [s1, s2, jnp.zeros((6, D), jnp.float32)], axis=0)

            @pl.when(i == 0)
            def _():
                st_ref[...] = blk

            @pl.when(i != 0)
            def _():
                st_ref[...] += blk

        @pl.when(p == 1)
        def _():
            m = st_ref[0:1, :] * (1.0 / N)
            v = st_ref[1:2, :] * (1.0 / N) - m * m
            h = (conv - m) * lax.rsqrt(v + EPS) * g_ref[...] + be_ref[...]
            h = jnp.maximum(h, 0.0)
            h = jnp.where(valid, h, 0.0)
            r = jnp.dot(h, w_ref[...],
                        preferred_element_type=jnp.float32) * dinv
            o_ref[0] = r[:, :DH]
            o_ref[1] = r[:, DH:]

    return pl.pallas_call(
        body,
        grid=(2, NPAD // BR),
        in_specs=[
            pl.BlockSpec((2, BR, DH), lambda p, i: (0, i, 0)),
            pl.BlockSpec((BR, DEGW), lambda p, i: (i, 0)),
            pl.BlockSpec((1, D), lambda p, i: (0, 0)),
            pl.BlockSpec((1, D), lambda p, i: (0, 0)),
            pl.BlockSpec((1, D), lambda p, i: (0, 0)),
            pl.BlockSpec((D, D), lambda p, i: (0, 0)),
        ],
        out_specs=pl.BlockSpec((2, BR, DH), lambda p, i: (0, i, 0)),
        out_shape=jax.ShapeDtypeStruct((2, NPAD, DH), jnp.float32),
        scratch_shapes=[pltpu.VMEM((8, D), jnp.float32)],
    )(a3, degw, b, g, be, w)


def _final(a3, degw, b):
    def body(a_ref, dg_ref, b_ref, o_ref):
        dinv = lax.rsqrt(dg_ref[:, 0:1])
        a = jnp.concatenate([a_ref[0], a_ref[1]], axis=1)
        o_ref[...] = a * dinv + b_ref[...]

    return pl.pallas_call(
        body,
        grid=(N // BRS,),
        in_specs=[
            pl.BlockSpec((2, BRS, DH), lambda i: (0, i, 0)),
            pl.BlockSpec((BRS, DEGW), lambda i: (i, 0)),
            pl.BlockSpec((1, D), lambda i: (0, 0)),
        ],
        out_specs=pl.BlockSpec((BRS, D), lambda i: (i, 0)),
        out_shape=jax.ShapeDtypeStruct((N, D), jnp.float32),
    )(a3, degw, b)


def kernel(x, edge_index_csr, W1, b1, g1, be1, W2, b2, g2, be2, W3, b3):
    src = edge_index_csr[0]
    dst = edge_index_csr[1]
    src2 = src.reshape(16, EPT)
    dst2 = dst.reshape(16, EPT)

    padd = DBLK * BLK - EPT
    pad_dd = jnp.broadcast_to(
        NPAD + (jnp.arange(padd, dtype=jnp.int32) & 127), (16, padd))
    dstd = jnp.concatenate([dst2, pad_dd], axis=1).reshape(16, DBLK, BLK)

    head = jnp.arange(RFIRST, dtype=jnp.int32)
    tailn = DBLKA * BLKA - RFIRST - EPT
    tail = jnp.arange(tailn, dtype=jnp.int32)
    hs = jnp.broadcast_to(N + (head % (NPAD - N)), (16, RFIRST))
    ts = jnp.broadcast_to(N + (tail % (NPAD - N)), (16, tailn))
    hd = jnp.broadcast_to((head * 79) % N, (16, RFIRST))
    td = jnp.broadcast_to((tail * 79) % N, (16, tailn))
    src2a = jnp.concatenate([hs, src2, ts], axis=1)
    dst2a = jnp.concatenate([hd, dst2, td], axis=1)
    src3 = jnp.concatenate([src2a, src2a + NPAD]).reshape(32, DBLKA, BLKA)
    dst3 = dst2a.reshape(16, DBLKA, BLKA)
    x_p = jnp.pad(x, ((0, NPAD - N), (0, 0)))
    ones = jnp.ones((NPAD, DEGW), jnp.float32)
    b1r, g1r, be1r = b1[None, :], g1[None, :], be1[None, :]
    b2r, g2r, be2r = b2[None, :], g2[None, :], be2[None, :]
    b3r = b3[None, :]

    degw = _deg(dstd, ones)
    p = _pre(x_p, W1, degw)
    a = jnp.reshape(_agg(jnp.reshape(p, (2 * NPAD, DH)), src3, dst3),
                    (2, NPAD, DH))
    p = _bn_apply(a, degw, b1r, g1r, be1r, W2)
    a = jnp.reshape(_agg(jnp.reshape(p, (2 * NPAD, DH)), src3, dst3),
                    (2, NPAD, DH))
    p = _bn_apply(a, degw, b2r, g2r, be2r, W3)
    a = jnp.reshape(_agg(jnp.reshape(p, (2 * NPAD, DH)), src3, dst3),
                    (2, NPAD, DH))
    return _final(a, degw, b3r)


# speedup vs baseline: 1.0107x; 1.0107x over previous
import functools

import jax
import jax.numpy as jnp
from jax import lax
from jax.experimental import pallas as pl
from jax.experimental.pallas import tpu as pltpu
from jax.experimental.pallas import tpu_sc as plsc

N = 10000
D = 256
DH = 128
NPAD = 10240
ACC = NPAD + 128
E = 160000
EPT = E // 16
TIR = NPAD // 16
BLK = 128
DBLK = 82
BLKA = 88
NP = 122
DBLKA = NP + 4
RFIRST = 2 * BLKA
DEGW = 16
EPS = 1e-5
BR = 2048
BRS = 2000


def _sc_mesh():
    return plsc.VectorSubcoreMesh(core_axis_name="c", subcore_axis_name="s")


def _deg(dst3, ones):
    @functools.partial(
        pl.kernel,
        out_type=jax.ShapeDtypeStruct((NPAD, DEGW), jnp.float32),
        mesh=_sc_mesh(),
        scratch_types=[
            pltpu.VMEM((DBLK, BLK), jnp.int32),
            pltpu.VMEM((BLK, DEGW), jnp.float32),
            pltpu.VMEM_SHARED((ACC, DEGW), jnp.float32),
            pltpu.SemaphoreType.DMA,
        ],
    )
    def k(dst_hbm, ones_hbm, out_hbm, dsts, ones_v, acc, sem):
        c = lax.axis_index("c")
        s = lax.axis_index("s")
        pltpu.sync_copy(dst_hbm.at[s], dsts)
        pltpu.sync_copy(ones_hbm.at[pl.ds(0, BLK)], ones_v)
        pltpu.sync_copy(ones_hbm.at[pl.ds(s * TIR, TIR)],
                        acc.at[pl.ds(s * TIR, TIR)])
        plsc.subcore_barrier()

        def body(j, carry):
            pltpu.sync_copy(ones_v, acc.at[dsts.at[j]], add=True)
            return carry

        lax.fori_loop(0, DBLK, body, 0)
        plsc.subcore_barrier()
        hh = NPAD // 2
        pltpu.sync_copy(acc.at[pl.ds(c * hh + s * (hh // 16), hh // 16)],
                        out_hbm.at[pl.ds(c * hh + s * (hh // 16), hh // 16)])

    return k(dst3, ones)


def _agg(p2, src3, dst3):
    @functools.partial(
        pl.kernel,
        out_type=jax.ShapeDtypeStruct((2 * NPAD, DH), jnp.float32),
        mesh=_sc_mesh(),
        scratch_types=[
            pltpu.VMEM((8, BLKA), jnp.int32),
            pltpu.VMEM((8, BLKA), jnp.int32),
            pltpu.VMEM((4, BLKA, DH), jnp.float32),
            pltpu.VMEM_SHARED((NPAD, DH), jnp.float32),
            pltpu.SemaphoreType.DMA,
            pltpu.SemaphoreType.DMA,
            pltpu.SemaphoreType.DMA,
            pltpu.SemaphoreType.DMA,
        ],
    )
    def k(p_hbm, src_hbm, dst_hbm, out_hbm, sidx, didx, rows, acc,
          sem_g, sem_s, sem_d, sem_c):
        c = lax.axis_index("c")
        s = lax.axis_index("s")
        w = c * 16 + s
        pltpu.sync_copy(p_hbm.at[pl.ds(c * NPAD + s * TIR, TIR)],
                        acc.at[pl.ds(s * TIR, TIR)])
        plsc.subcore_barrier()

        def load_idx(j, sl):
            pltpu.async_copy(src_hbm.at[w, j], sidx.at[sl], sem_s)
            pltpu.async_copy(dst_hbm.at[s, j], didx.at[sl], sem_d)

        def gather(sl, rb):
            pltpu.async_copy(p_hbm.at[sidx.at[sl]], rows.at[rb], sem_g)

        def scatter(rb, sl):
            pltpu.async_copy(rows.at[rb], acc.at[didx.at[sl]], sem_c,
                             add=True)

        def wait_idx():
            pltpu.make_async_copy(src_hbm.at[0, 0], sidx.at[0], sem_s).wait()
            pltpu.make_async_copy(dst_hbm.at[0, 0], didx.at[0], sem_d).wait()

        def wait_gather():
            pltpu.make_async_copy(p_hbm.at[pl.ds(0, BLKA)], rows.at[0],
                                  sem_g).wait()

        def wait_scatter():
            pltpu.make_async_copy(p_hbm.at[pl.ds(0, BLKA)], rows.at[0],
                                  sem_c).wait()

        for j in range(4):
            pltpu.sync_copy(src_hbm.at[w, j], sidx.at[j])
            pltpu.sync_copy(dst_hbm.at[s, j], didx.at[j])
        gather(0, 0)
        gather(1, 1)
        load_idx(4, 4)
        load_idx(5, 5)
        wait_gather()
        gather(2, 2)
        scatter(0, 0)
        load_idx(6, 6)
        wait_gather()
        gather(3, 3)
        scatter(1, 1)
        load_idx(7, 7)

        def body(kk, carry):
            u0 = 2 + kk * 8
            for t in range(8):
                u = u0 + t
                wait_gather()
                wait_idx()
                wait_scatter()
                gather((u + 2) % 8, (u + 2) % 4)
                scatter(u % 4, u % 8)
                load_idx(u + 4, (u + 4) % 8)
            return carry

        lax.fori_loop(0, (NP - 2) // 8, body, 0)
        for _ in range(2):
            wait_gather()
            wait_idx()
            wait_scatter()
        plsc.subcore_barrier()
        pltpu.sync_copy(acc.at[pl.ds(s * TIR, TIR)],
                        out_hbm.at[pl.ds(c * NPAD + s * TIR, TIR)])

    return k(p2, src3, dst3)


def _pre(x_p, w, degw):
    def body(x_ref, w_ref, dg_ref, o_ref):
        dinv = lax.rsqrt(dg_ref[:, 0:1])
        o_ref[0] = jnp.dot(x_ref[...], w_ref[...],
                           preferred_element_type=jnp.float32) * dinv

    return pl.pallas_call(
        body,
        grid=(2, NPAD // BR),
        in_specs=[
            pl.BlockSpec((BR, D), lambda c, i: (i, 0)),
            pl.BlockSpec((D, DH), lambda c, i: (0, c)),
            pl.BlockSpec((BR, DEGW), lambda c, i: (i, 0)),
        ],
        out_specs=pl.BlockSpec((1, BR, DH), lambda c, i: (c, i, 0)),
        out_shape=jax.ShapeDtypeStruct((2, NPAD, DH), jnp.float32),
    )(x_p, w, degw)


def _bn_apply(a3, degw, b, g, be, w):
    def body(a_ref, dg_ref, b_ref, g_ref, be_ref, w_ref, o_ref, st_ref):
        p = pl.program_id(0)
        i = pl.program_id(1)
        dinv = lax.rsqrt(dg_ref[:, 0:1])
        a = jnp.concatenate([a_ref[0], a_ref[1]], axis=1)
        conv = a * dinv + b_ref[...]
        rid = i * BR + lax.broadcasted_iota(jnp.int32, (BR, 1), 0)
        valid = rid < N

        @pl.when(p == 0)
        def _():
            cm = jnp.where(valid, conv, 0.0)
            s1 = jnp.sum(cm, axis=0, keepdims=True)
            s2 = jnp.sum(cm * cm, axis=0, keepdims=True)
            blk = jnp.concatenate(
                [s1, s2, jnp.zeros((6, D), jnp.float32)], axis=0)

            @pl.when(i == 0)
            def _():
                st_ref[...] = blk

            @pl.when(i != 0)
            def _():
                st_ref[...] += blk

        @pl.when(p == 1)
        def _():
            m = st_ref[0:1, :] * (1.0 / N)
            v = st_ref[1:2, :] * (1.0 / N) - m * m
            h = (conv - m) * lax.rsqrt(v + EPS) * g_ref[...] + be_ref[...]
            h = jnp.maximum(h, 0.0)
            h = jnp.where(valid, h, 0.0)
            r = jnp.dot(h, w_ref[...],
                        preferred_element_type=jnp.float32) * dinv
            o_ref[0] = r[:, :DH]
            o_ref[1] = r[:, DH:]

    return pl.pallas_call(
        body,
        grid=(2, NPAD // BR),
        in_specs=[
            pl.BlockSpec((2, BR, DH), lambda p, i: (0, i, 0)),
            pl.BlockSpec((BR, DEGW), lambda p, i: (i, 0)),
            pl.BlockSpec((1, D), lambda p, i: (0, 0)),
            pl.BlockSpec((1, D), lambda p, i: (0, 0)),
            pl.BlockSpec((1, D), lambda p, i: (0, 0)),
            pl.BlockSpec((D, D), lambda p, i: (0, 0)),
        ],
        out_specs=pl.BlockSpec((2, BR, DH), lambda p, i: (0, p * i, 0)),
        out_shape=jax.ShapeDtypeStruct((2, NPAD, DH), jnp.float32),
        scratch_shapes=[pltpu.VMEM((8, D), jnp.float32)],
    )(a3, degw, b, g, be, w)


def _final(a3, degw, b):
    def body(a_ref, dg_ref, b_ref, o_ref):
        dinv = lax.rsqrt(dg_ref[:, 0:1])
        a = jnp.concatenate([a_ref[0], a_ref[1]], axis=1)
        o_ref[...] = a * dinv + b_ref[...]

    return pl.pallas_call(
        body,
        grid=(N // BRS,),
        in_specs=[
            pl.BlockSpec((2, BRS, DH), lambda i: (0, i, 0)),
            pl.BlockSpec((BRS, DEGW), lambda i: (i, 0)),
            pl.BlockSpec((1, D), lambda i: (0, 0)),
        ],
        out_specs=pl.BlockSpec((BRS, D), lambda i: (i, 0)),
        out_shape=jax.ShapeDtypeStruct((N, D), jnp.float32),
    )(a3, degw, b)


def kernel(x, edge_index_csr, W1, b1, g1, be1, W2, b2, g2, be2, W3, b3):
    src = edge_index_csr[0]
    dst = edge_index_csr[1]
    src2 = src.reshape(16, EPT)
    dst2 = dst.reshape(16, EPT)

    padd = DBLK * BLK - EPT
    pad_dd = jnp.broadcast_to(
        NPAD + (jnp.arange(padd, dtype=jnp.int32) & 127), (16, padd))
    dstd = jnp.concatenate([dst2, pad_dd], axis=1).reshape(16, DBLK, BLK)

    head = jnp.arange(RFIRST, dtype=jnp.int32)
    tailn = DBLKA * BLKA - RFIRST - EPT
    tail = jnp.arange(tailn, dtype=jnp.int32)
    hs = jnp.broadcast_to(N + (head % (NPAD - N)), (16, RFIRST))
    ts = jnp.broadcast_to(N + (tail % (NPAD - N)), (16, tailn))
    hd = jnp.broadcast_to((head * 79) % N, (16, RFIRST))
    td = jnp.broadcast_to((tail * 79) % N, (16, tailn))
    src2a = jnp.concatenate([hs, src2, ts], axis=1)
    dst2a = jnp.concatenate([hd, dst2, td], axis=1)
    src3 = jnp.concatenate([src2a, src2a + NPAD]).reshape(32, DBLKA, BLKA)
    dst3 = dst2a.reshape(16, DBLKA, BLKA)
    x_p = jnp.pad(x, ((0, NPAD - N), (0, 0)))
    ones = jnp.ones((NPAD, DEGW), jnp.float32)
    b1r, g1r, be1r = b1[None, :], g1[None, :], be1[None, :]
    b2r, g2r, be2r = b2[None, :], g2[None, :], be2[None, :]
    b3r = b3[None, :]

    degw = _deg(dstd, ones)
    p = _pre(x_p, W1, degw)
    a = jnp.reshape(_agg(jnp.reshape(p, (2 * NPAD, DH)), src3, dst3),
                    (2, NPAD, DH))
    p = _bn_apply(a, degw, b1r, g1r, be1r, W2)
    a = jnp.reshape(_agg(jnp.reshape(p, (2 * NPAD, DH)), src3, dst3),
                    (2, NPAD, DH))
    p = _bn_apply(a, degw, b2r, g2r, be2r, W3)
    a = jnp.reshape(_agg(jnp.reshape(p, (2 * NPAD, DH)), src3, dst3),
                    (2, NPAD, DH))
    return _final(a, degw, b3r)
